# baseline (speedup 1.0000x reference)
import functools

import jax
import jax.numpy as jnp
from jax import lax
from jax.experimental import pallas as pl
from jax.experimental.pallas import tpu as pltpu
from jax.experimental.pallas import tpu_sc as plsc

D = 64
DP = 128
SEQ = 200
BATCH = 4096


def _make_kernel(num_workers):
    blk = BATCH // num_workers
    mesh = plsc.VectorSubcoreMesh(core_axis_name="c", subcore_axis_name="s")

    @functools.partial(
        pl.kernel,
        out_type=jax.ShapeDtypeStruct((SEQ, D, BATCH), jnp.float32),
        mesh=mesh,
        scratch_types=[
            pltpu.VMEM((blk * SEQ,), jnp.int32),
            pltpu.VMEM((SEQ, DP), jnp.float32),
            pltpu.VMEM((blk,), jnp.int32),
            pltpu.VMEM((blk,), jnp.int32),
            pltpu.VMEM((blk, DP), jnp.float32),
            pltpu.VMEM((blk, DP), jnp.float32),
            pltpu.VMEM((D, blk), jnp.float32),
            pltpu.VMEM((D, blk), jnp.float32),
            pltpu.SemaphoreType.DMA,
            pltpu.SemaphoreType.DMA,
            pltpu.SemaphoreType.DMA,
            pltpu.SemaphoreType.DMA,
        ],
        compiler_params=pltpu.CompilerParams(
            use_tc_tiling_on_sc=True, needs_layout_passes=False),
    )
    def body(idx_hbm, tbl_hbm, pos_hbm, out_hbm, idxb, pos_v,
             col0, col1, tok0, tok1, tr0, tr1, gs0, gs1, ws0, ws1):
        nc = 2
        wid = lax.axis_index("s") * nc + lax.axis_index("c")
        b0 = wid * blk
        pltpu.sync_copy(idx_hbm.at[pl.ds(b0 * SEQ, blk * SEQ)], idxb)
        pltpu.sync_copy(pos_hbm, pos_v)

        lane = jnp.arange(16, dtype=jnp.int32)
        bs200 = [lane * SEQ + g * 16 * SEQ for g in range(blk // 16)]
        bvec = [lane + g * 16 for g in range(blk // 16)]

        def extract_col(s, col):
            sv = jnp.full((16,), s, jnp.int32)
            for g in range(blk // 16):
                col[pl.ds(g * 16, 16)] = plsc.load_gather(
                    idxb, [bs200[g] + sv])

        def start_gather(col, tok, sem):
            pltpu.async_copy(tbl_hbm.at[col], tok, sem)

        def wait_gather(col, tok, sem):
            pltpu.make_async_copy(tbl_hbm.at[col], tok, sem).wait()

        def out_slice(s):
            return out_hbm.at[s, :, pl.ds(b0, blk)]

        fvecs = [lane + j * 16 for j in range(D // 16)]

        def transpose_add(s, tok, tr):
            pv = [pos_v[s, pl.ds(j * 16, 16)] for j in range(D // 16)]

            @plsc.parallel_loop(0, blk, unroll=16)
            def bbody(b):
                bs = jnp.full((16,), b, jnp.int32)
                for j in range(D // 16):
                    x = tok[b, pl.ds(j * 16, 16)] + pv[j]
                    plsc.store_scatter(tr, [fvecs[j], bs], x)

        extract_col(0, col0)
        start_gather(col0, tok0, gs0)

        bufs = ((col0, tok0, tr0, gs0, ws0), (col1, tok1, tr1, gs1, ws1))

        def step(j, carry):
            for b in range(2):
                s = 2 * j + b
                col, tok, tr, gs, ws = bufs[b]
                ncol, ntok, _, ngs, _ = bufs[1 - b]
                nxt_ok = (s + 1 < SEQ) if b == 0 else None
                if b == 0:
                    extract_col(s + 1, ncol)
                    start_gather(ncol, ntok, ngs)
                else:
                    @pl.when(j < SEQ // 2 - 1)
                    def _():
                        extract_col(s + 1, ncol)
                        start_gather(ncol, ntok, ngs)
                wait_gather(col, tok, gs)

                @pl.when(j >= 1)
                def _():
                    pltpu.make_async_copy(tr, out_slice(s), ws).wait()

                transpose_add(s, tok, tr)
                pltpu.async_copy(tr, out_slice(s), ws)
            return carry

        lax.fori_loop(0, SEQ // 2, step, 0)
        pltpu.make_async_copy(tr0, out_slice(0), ws0).wait()
        pltpu.make_async_copy(tr1, out_slice(1), ws1).wait()

    return body


def kernel(inputs, token_table, pos_table):
    idx_flat = inputs.reshape(-1).astype(jnp.int32)
    tblp = jnp.pad(token_table, ((0, 0), (0, DP - D)))
    posp = jnp.pad(pos_table, ((0, 0), (0, DP - D)))
    info = plsc.get_sparse_core_info()
    nw = info.num_cores * info.num_subcores
    out_t = _make_kernel(nw)(idx_flat, tblp, posp)
    return out_t.transpose(2, 0, 1)


# speedup vs baseline: 1.0154x; 1.0154x over previous
import functools

import jax
import jax.numpy as jnp
from jax import lax
from jax.experimental import pallas as pl
from jax.experimental.pallas import tpu as pltpu
from jax.experimental.pallas import tpu_sc as plsc

D = 64
DP = 128
SEQ = 200
BATCH = 4096


def _make_kernel(num_workers):
    blk = BATCH // num_workers
    mesh = plsc.VectorSubcoreMesh(core_axis_name="c", subcore_axis_name="s")

    @functools.partial(
        pl.kernel,
        out_type=jax.ShapeDtypeStruct((SEQ, D, BATCH), jnp.float32),
        mesh=mesh,
        scratch_types=[
            pltpu.VMEM((blk * SEQ,), jnp.int32),
            pltpu.VMEM((SEQ, DP), jnp.float32),
            pltpu.VMEM((blk,), jnp.int32),
            pltpu.VMEM((blk,), jnp.int32),
            pltpu.VMEM((blk, DP), jnp.float32),
            pltpu.VMEM((blk, DP), jnp.float32),
            pltpu.VMEM((D, blk), jnp.float32),
            pltpu.VMEM((D, blk), jnp.float32),
            pltpu.SemaphoreType.DMA,
            pltpu.SemaphoreType.DMA,
            pltpu.SemaphoreType.DMA,
            pltpu.SemaphoreType.DMA,
        ],
        compiler_params=pltpu.CompilerParams(
            use_tc_tiling_on_sc=True, needs_layout_passes=False),
    )
    def body(idx_hbm, tbl_hbm, pos_hbm, out_hbm, idxb, pos_v,
             col0, col1, tok0, tok1, tr0, tr1, gs0, gs1, ws0, ws1):
        nc = 2
        wid = lax.axis_index("s") * nc + lax.axis_index("c")
        b0 = wid * blk
        pltpu.sync_copy(idx_hbm.at[pl.ds(b0 * SEQ, blk * SEQ)], idxb)
        pltpu.sync_copy(pos_hbm, pos_v)

        lane = jnp.arange(16, dtype=jnp.int32)
        bs200 = [lane * SEQ + g * 16 * SEQ for g in range(blk // 16)]
        bvec = [lane + g * 16 for g in range(blk // 16)]

        def extract_col(s, col):
            sv = jnp.full((16,), s, jnp.int32)
            for g in range(blk // 16):
                col[pl.ds(g * 16, 16)] = plsc.load_gather(
                    idxb, [bs200[g] + sv])

        def start_gather(col, tok, sem):
            pltpu.async_copy(tbl_hbm.at[col], tok, sem)

        def wait_gather(col, tok, sem):
            pltpu.make_async_copy(tbl_hbm.at[col], tok, sem).wait()

        def out_slice(s):
            return out_hbm.at[s, :, pl.ds(b0, blk)]

        fvecs = [lane + j * 16 for j in range(D // 16)]

        def transpose_add(s, tok, tr):
            sv = jnp.full((16,), s, jnp.int32)

            @plsc.parallel_loop(0, D, unroll=8)
            def fbody(f):
                fv = jnp.full((16,), f, jnp.int32)
                pvf = plsc.load_gather(pos_v, [sv, fv])
                for g in range(blk // 16):
                    x = plsc.load_gather(tok, [bvec[g], fv])
                    tr[f, pl.ds(g * 16, 16)] = x + pvf

        extract_col(0, col0)
        start_gather(col0, tok0, gs0)

        bufs = ((col0, tok0, tr0, gs0, ws0), (col1, tok1, tr1, gs1, ws1))

        def step(j, carry):
            for b in range(2):
                s = 2 * j + b
                col, tok, tr, gs, ws = bufs[b]
                ncol, ntok, _, ngs, _ = bufs[1 - b]
                nxt_ok = (s + 1 < SEQ) if b == 0 else None
                if b == 0:
                    extract_col(s + 1, ncol)
                    start_gather(ncol, ntok, ngs)
                else:
                    @pl.when(j < SEQ // 2 - 1)
                    def _():
                        extract_col(s + 1, ncol)
                        start_gather(ncol, ntok, ngs)
                wait_gather(col, tok, gs)

                @pl.when(j >= 1)
                def _():
                    pltpu.make_async_copy(tr, out_slice(s), ws).wait()

                transpose_add(s, tok, tr)
                pltpu.async_copy(tr, out_slice(s), ws)
            return carry

        lax.fori_loop(0, SEQ // 2, step, 0)
        pltpu.make_async_copy(tr0, out_slice(0), ws0).wait()
        pltpu.make_async_copy(tr1, out_slice(1), ws1).wait()

    return body


def kernel(inputs, token_table, pos_table):
    idx_flat = inputs.reshape(-1).astype(jnp.int32)
    tblp = jnp.pad(token_table, ((0, 0), (0, DP - D)))
    posp = jnp.pad(pos_table, ((0, 0), (0, DP - D)))
    info = plsc.get_sparse_core_info()
    nw = info.num_cores * info.num_subcores
    out_t = _make_kernel(nw)(idx_flat, tblp, posp)
    return out_t.transpose(2, 0, 1)
